# baseline (speedup 1.0000x reference)
import functools

import jax
import jax.numpy as jnp
from jax import lax
from jax.experimental import pallas as pl
from jax.experimental.pallas import tpu as pltpu
from jax.experimental.pallas import tpu_sc as plsc

B = 64
H = 64

K = 128
SB = 7
NSUB = 16
NCORE = 2

F32 = jnp.float32

_SC_PARAMS = pltpu.CompilerParams(use_tc_tiling_on_sc=False)


def _f32(shape):
    return jax.ShapeDtypeStruct(shape, F32)


def _ce_body(ea8_ref, w8_ref, b8_ref, o0, o1, o2):
    ea = ea8_ref[...]
    outs = (o0, o1, o2)
    for l in range(3):
        for qq in range(4):
            r = jnp.dot(ea, w8_ref[l, qq], preferred_element_type=F32)
            outs[l][qq] = r + b8_ref[l, qq]


def _ce_call(ea8, w8, b8):
    e8 = ea8.shape[0]
    eb8 = 512
    grid = e8 // eb8
    out_spec = pl.BlockSpec((4, eb8, 128), lambda i: (0, i, 0))
    return pl.pallas_call(
        _ce_body,
        grid=(grid,),
        in_specs=[
            pl.BlockSpec((eb8, 24), lambda i: (i, 0)),
            pl.BlockSpec((3, 4, 24, 128), lambda i: (0, 0, 0, 0)),
            pl.BlockSpec((3, 4, 1, 128), lambda i: (0, 0, 0, 0)),
        ],
        out_specs=[out_spec, out_spec, out_spec],
        out_shape=[_f32((4, e8, 128))] * 3,
    )(ea8, w8, b8)


def _hwf_body(x_ref, w_ref, o_ref):
    o_ref[...] = jnp.dot(x_ref[...], w_ref[...], preferred_element_type=F32)


def _hwf_call(x, wh0, rb):
    n = x.shape[0]
    return pl.pallas_call(
        _hwf_body,
        grid=(n // rb,),
        in_specs=[
            pl.BlockSpec((rb, x.shape[1]), lambda i: (i, 0)),
            pl.BlockSpec(wh0.shape, lambda i: (0, 0)),
        ],
        out_specs=pl.BlockSpec((rb, H), lambda i: (i, 0)),
        out_shape=_f32((n, H)),
    )(x, wh0)


def _split_quarters(hwf):
    n = hwf.shape[0]
    return jnp.transpose(hwf.reshape(n, 4, 16), (1, 0, 2)).reshape(4 * n, 16)


def _edge_sc_call(hw4, ce, srcf, dstf, n, nrows):
    total_blocks = srcf.shape[0] // K
    blocks_pw = total_blocks // NSUB
    rows_pw = nrows // NSUB
    nsuper = blocks_pw // SB
    mesh = plsc.VectorSubcoreMesh(core_axis_name="c", subcore_axis_name="s")

    @functools.partial(
        pl.kernel,
        mesh=mesh,
        compiler_params=_SC_PARAMS,
        out_type=_f32((4, nrows, 16)),
        scratch_types=[
            pltpu.VMEM((4, 1, SB * K), jnp.int32),
            pltpu.VMEM((4, 1, SB * K), jnp.int32),
            pltpu.VMEM((2, SB * K, 16), F32),
            pltpu.VMEM((2, SB * (K // 8), 128), F32),
            pltpu.VMEM_SHARED((nrows, 16), F32),
            pltpu.SemaphoreType.DMA,
            pltpu.SemaphoreType.DMA,
            pltpu.SemaphoreType.DMA,
            pltpu.SemaphoreType.DMA,
        ],
    )
    def k(hw_hbm, ce_hbm, srcf_hbm, dstf_hbm, out_hbm, sidx, didx, gbuf, cbuf,
          acc, semi, semg, semc, sems2):
        c = lax.axis_index("c")
        s = lax.axis_index("s")
        zero = jnp.zeros((16,), F32)
        base_blk = s * blocks_pw

        for qq in range(2):
            q_id = 2 * qq + c
            qn = q_id * n

            @pl.loop(0, K)
            def _(r):
                gbuf[0, r, pl.ds(0, 16)] = zero

            @pl.loop(0, rows_pw // K)
            def _(kk):
                pltpu.sync_copy(gbuf.at[0, pl.ds(0, K)],
                                acc.at[pl.ds(s * rows_pw + kk * K, K)])

            plsc.subcore_barrier()

            def issue_idx(ob):
                m = ob % 4
                b0 = base_blk + ob * SB
                pltpu.async_copy(srcf_hbm.at[pl.ds(b0 * K, SB * K)],
                                 sidx.at[m, 0], semi)
                pltpu.async_copy(dstf_hbm.at[pl.ds(b0 * K, SB * K)],
                                 didx.at[m, 0], semi)

            def drain_and_offset_idx(m):
                pltpu.make_async_copy(srcf_hbm.at[pl.ds(0, SB * K)],
                                      sidx.at[0, 0], semi).wait()
                pltpu.make_async_copy(dstf_hbm.at[pl.ds(0, SB * K)],
                                      didx.at[0, 0], semi).wait()
                @pl.loop(0, SB * K // 16)
                def _(t):
                    v = sidx[m, 0, pl.ds(16 * t, 16)]
                    sidx[m, 0, pl.ds(16 * t, 16)] = v + qn

            def fire(ob):
                q = ob % 2
                m = ob % 4
                b0 = base_blk + ob * SB
                pltpu.async_copy(hw_hbm.at[sidx.at[m, 0]], gbuf.at[q], semg)
                pltpu.async_copy(
                    ce_hbm.at[q_id, pl.ds(b0 * (K // 8), SB * (K // 8))],
                    cbuf.at[q], semc)

            def drain_fire(q, m):
                pltpu.make_async_copy(hw_hbm.at[sidx.at[m, 0]],
                                      gbuf.at[q], semg).wait()
                pltpu.make_async_copy(
                    ce_hbm.at[0, pl.ds(0, SB * (K // 8))],
                    cbuf.at[q], semc).wait()

            def drain_scatter(q, m):
                pltpu.make_async_copy(gbuf.at[q], acc.at[didx.at[m, 0]],
                                      sems2).wait()

            issue_idx(0)
            drain_and_offset_idx(0)
            issue_idx(1)
            fire(0)

            @pl.loop(0, nsuper)
            def _(ob):
                q = ob % 2
                m = ob % 4
                drain_fire(q, m)
                for j in range(SB):
                    @pl.loop(0, K // 8)
                    def _(rr):
                        for sub in range(8):
                            v = (gbuf[q, j * K + 8 * rr + sub, pl.ds(0, 16)]
                                 + cbuf[q, j * (K // 8) + rr,
                                        pl.ds(16 * sub, 16)])
                            gbuf[q, j * K + 8 * rr + sub, pl.ds(0, 16)] = (
                                jnp.maximum(v, 0.0))

                pltpu.async_copy(gbuf.at[q], acc.at[didx.at[m, 0]], sems2,
                                 add=True)

                @pl.when(ob > 0)
                def _():
                    drain_scatter((ob + 1) % 2, (ob + 3) % 4)

                @pl.when(ob < nsuper - 1)
                def _():
                    drain_and_offset_idx((ob + 1) % 4)
                    fire(ob + 1)

                @pl.when(ob < nsuper - 2)
                def _():
                    issue_idx(ob + 2)

            drain_scatter((nsuper - 1) % 2, (nsuper - 1) % 4)
            plsc.subcore_barrier()
            pltpu.sync_copy(acc.at[pl.ds(s * rows_pw, rows_pw)],
                            out_hbm.at[q_id, pl.ds(s * rows_pw, rows_pw)])
            plsc.subcore_barrier()

    return k(hw4, ce, srcf, dstf)


def _upd_compute(h_ref, a_refs, uh_ref, u_refs, bias_ref):
    acc = jnp.dot(h_ref[...], uh_ref[...], preferred_element_type=F32)
    for a_ref, u_ref in zip(a_refs, u_refs):
        acc = acc + jnp.dot(a_ref[...], u_ref[...], preferred_element_type=F32)
    return jnp.maximum(acc + bias_ref[...], 0.0)


def _upd_body_hw(h_ref, a0, a1, a2, a3, uh_ref, u0, u1, u2, u3, bias_ref,
                 whn_ref, ho_ref, hwo_ref):
    hn = _upd_compute(h_ref, (a0, a1, a2, a3), uh_ref, (u0, u1, u2, u3),
                      bias_ref)
    ho_ref[...] = hn
    hwo_ref[...] = jnp.dot(hn, whn_ref[...], preferred_element_type=F32)


def _upd_body(h_ref, a0, a1, a2, a3, uh_ref, u0, u1, u2, u3, bias_ref,
              ho_ref):
    ho_ref[...] = _upd_compute(h_ref, (a0, a1, a2, a3), uh_ref,
                               (u0, u1, u2, u3), bias_ref)


def _upd_call(h, aggs, uh, us, bias, whn, rb):
    n = h.shape[0]
    dh = h.shape[1]
    base_in = ([pl.BlockSpec((rb, dh), lambda i: (i, 0))]
               + [pl.BlockSpec((rb, 16), lambda i: (i, 0))] * 4
               + [pl.BlockSpec(uh.shape, lambda i: (0, 0))]
               + [pl.BlockSpec((16, H), lambda i: (0, 0))] * 4
               + [pl.BlockSpec((1, H), lambda i: (0, 0))])
    args = [h, *aggs, uh, *us, bias]
    if whn is not None:
        return pl.pallas_call(
            _upd_body_hw,
            grid=(n // rb,),
            in_specs=base_in + [pl.BlockSpec((H, H), lambda i: (0, 0))],
            out_specs=[
                pl.BlockSpec((rb, H), lambda i: (i, 0)),
                pl.BlockSpec((rb, H), lambda i: (i, 0)),
            ],
            out_shape=[_f32((n, H)), _f32((n, H))],
        )(*args, whn)
    return pl.pallas_call(
        _upd_body,
        grid=(n // rb,),
        in_specs=base_in,
        out_specs=pl.BlockSpec((rb, H), lambda i: (i, 0)),
        out_shape=_f32((n, H)),
    )(*args)


def _pool_body(h_ref, x_ref, batch_ref, sum_ref, cnt_ref, max_ref, last_ref,
               *, rb):
    i = pl.program_id(0)

    @pl.when(i == 0)
    def _():
        sum_ref[...] = jnp.zeros_like(sum_ref)
        cnt_ref[...] = jnp.zeros_like(cnt_ref)
        max_ref[...] = jnp.full_like(max_ref, -1e30)
        last_ref[...] = jnp.full_like(last_ref, -1)

    bcol = batch_ref[0]
    hb = h_ref[...]
    oh = (bcol == lax.broadcasted_iota(jnp.int32, (rb, B), 1)).astype(F32)
    sum_ref[...] += lax.dot_general(oh, hb, (((0,), (0,)), ((), ())),
                                    preferred_element_type=F32)
    cnt_ref[...] += lax.dot_general(oh, jnp.ones((rb, 8), F32),
                                    (((0,), (0,)), ((), ())),
                                    preferred_element_type=F32)

    def mbody(b, _):
        mask = bcol == b
        m = jnp.max(jnp.where(mask, hb, -1e30), axis=0, keepdims=True)
        cur = max_ref[pl.ds(b, 1), :]
        max_ref[pl.ds(b, 1), :] = jnp.maximum(cur, m)
        return 0

    lax.fori_loop(0, B, mbody, 0)

    xb = x_ref[...]
    best = xb[:, 0:1]
    nt = jnp.zeros_like(best, dtype=jnp.int32)
    for kf in range(1, 4):
        upd = xb[:, kf:kf + 1] > best
        nt = jnp.where(upd, kf, nt)
        best = jnp.maximum(best, xb[:, kf:kf + 1])

    idxg = i * rb + lax.broadcasted_iota(jnp.int32, (rb, B), 0)
    ohb = oh > 0.0
    for t in range(3):
        cond = (nt == t) & ohb
        last_t = jnp.max(jnp.where(cond, idxg, -1), axis=0, keepdims=True)
        cur = last_ref[pl.ds(t, 1), :]
        last_ref[pl.ds(t, 1), :] = jnp.maximum(cur, last_t)


def _pool_call(h, x, batch3, rb):
    n = h.shape[0]
    full = lambda shape: pl.BlockSpec(shape, lambda i: tuple(0 for _ in shape))
    return pl.pallas_call(
        functools.partial(_pool_body, rb=rb),
        grid=(n // rb,),
        in_specs=[
            pl.BlockSpec((rb, H), lambda i: (i, 0)),
            pl.BlockSpec((rb, 4), lambda i: (i, 0)),
            pl.BlockSpec((1, rb, 1), lambda i: (i, 0, 0)),
        ],
        out_specs=[full((B, H)), full((B, 8)), full((B, H)), full((8, B))],
        out_shape=[_f32((B, H)), _f32((B, 8)), _f32((B, H)),
                   jax.ShapeDtypeStruct((8, B), jnp.int32)],
    )(h, x, batch3)


def _gather_sc_call(h, idx):
    mesh = plsc.VectorSubcoreMesh(core_axis_name="c", subcore_axis_name="s")

    @functools.partial(
        pl.kernel,
        mesh=mesh,
        compiler_params=_SC_PARAMS,
        out_type=_f32((NCORE * NSUB * 8, H)),
        scratch_types=[
            pltpu.VMEM((8,), jnp.int32),
            pltpu.VMEM((8, H), F32),
            pltpu.SemaphoreType.DMA,
        ],
    )
    def k(h_hbm, idx_hbm, out_hbm, idxv, rows, sem):
        c = lax.axis_index("c")
        s = lax.axis_index("s")
        w = s * NCORE + c
        pltpu.sync_copy(idx_hbm.at[w], idxv)
        pltpu.async_copy(h_hbm.at[idxv], rows, sem).wait()
        pltpu.sync_copy(rows, out_hbm.at[pl.ds(w * 8, 8)])

    return k(h, idx)


def _heads1_body(sum_ref, cnt_ref, max_ref, last_ref, gath_ref,
                 w1a_ref, w1b_ref, b1_ref, w2_ref, b2_ref,
                 tmw_ref, tmb_ref, tlw_ref, tlb_ref,
                 vw_ref, vb_ref, vmw_ref, vmb_ref, vlw_ref, vlb_ref,
                 a1v_ref, ab1_ref,
                 mut_ref, lvt_ref, muv_ref, lvv_ref, hterm_ref, pre_ref):
    cnt = cnt_ref[:, 0:1]
    meanh = sum_ref[...] / jnp.maximum(cnt, 1.0)
    maxh = jnp.where(cnt > 0.0, max_ref[...], 0.0)
    ht = jnp.dot(meanh, w1a_ref[...], preferred_element_type=F32)
    ht = ht + jnp.dot(maxh, w1b_ref[...], preferred_element_type=F32)
    ht = jnp.maximum(ht + b1_ref[...], 0.0)
    ht = jnp.maximum(jnp.dot(ht, w2_ref[...], preferred_element_type=F32)
                     + b2_ref[...], 0.0)
    mut_ref[...] = jnp.dot(ht, tmw_ref[...], preferred_element_type=F32) + tmb_ref[...]
    lvt_ref[...] = jnp.dot(ht, tlw_ref[...], preferred_element_type=F32) + tlb_ref[...]

    terms = []
    for t in range(3):
        valid = last_ref[:, t:t + 1] >= 0
        terms.append(jnp.where(valid, gath_ref[t * B:(t + 1) * B, :], 0.0))
    hterm = jnp.concatenate(terms, axis=1)
    hterm_ref[...] = hterm
    hv = jnp.maximum(jnp.dot(hterm, vw_ref[...], preferred_element_type=F32)
                     + vb_ref[...], 0.0)
    muv_ref[...] = jnp.dot(hv, vmw_ref[...], preferred_element_type=F32) + vmb_ref[...]
    lvv_ref[...] = jnp.dot(hv, vlw_ref[...], preferred_element_type=F32) + vlb_ref[...]
    pre_ref[...] = jnp.dot(terms[1], a1v_ref[...], preferred_element_type=F32) + ab1_ref[...]


def _heads1_call(sumh, cnt, maxh, last, gath, wdict):
    ins = [sumh, cnt, maxh, last, gath,
           wdict['topo_W1a'], wdict['topo_W1b'], wdict['topo_b1'],
           wdict['topo_W2'], wdict['topo_b2'],
           wdict['topo_mu_W'], wdict['topo_mu_b'],
           wdict['topo_lv_W'], wdict['topo_lv_b'],
           wdict['val_W'], wdict['val_b'],
           wdict['val_mu_W'], wdict['val_mu_b'],
           wdict['val_lv_W'], wdict['val_lv_b'],
           wdict['attn_A1v'], wdict['attn_b1']]
    full = lambda a: pl.BlockSpec(a.shape, lambda: tuple(0 for _ in a.shape))
    return pl.pallas_call(
        _heads1_body,
        in_specs=[full(a) for a in ins],
        out_specs=[pl.BlockSpec((B, 2), lambda: (0, 0)),
                   pl.BlockSpec((B, 2), lambda: (0, 0)),
                   pl.BlockSpec((B, 2), lambda: (0, 0)),
                   pl.BlockSpec((B, 2), lambda: (0, 0)),
                   pl.BlockSpec((B, 3 * H), lambda: (0, 0)),
                   pl.BlockSpec((B, H), lambda: (0, 0))],
        out_shape=[_f32((B, 2)), _f32((B, 2)), _f32((B, 2)), _f32((B, 2)),
                   _f32((B, 3 * H)), _f32((B, H))],
    )(*ins)


def _attn1_body(h_ref, batch_ref, pre_ref, a1h_ref, a2_ref, b2_ref,
                sc_ref, smax_ref, *, rb):
    i = pl.program_id(0)

    @pl.when(i == 0)
    def _():
        smax_ref[...] = jnp.full_like(smax_ref, -1e30)

    bcol = batch_ref[0]
    hb = h_ref[...]
    oh = (bcol == lax.broadcasted_iota(jnp.int32, (rb, B), 1)).astype(F32)
    t1 = jnp.tanh(jnp.dot(hb, a1h_ref[...], preferred_element_type=F32)
                  + jnp.dot(oh, pre_ref[...], preferred_element_type=F32))
    s = jnp.dot(t1, a2_ref[...], preferred_element_type=F32) + b2_ref[...]
    sc_ref[...] = s
    sm = jnp.max(jnp.where(oh > 0.0, s, -1e30), axis=0, keepdims=True)
    smax_ref[0:1, :] = jnp.maximum(smax_ref[0:1, :], sm)


def _attn1_call(h, batch3, pre, a1h, a2, b2, rb):
    n = h.shape[0]
    return pl.pallas_call(
        functools.partial(_attn1_body, rb=rb),
        grid=(n // rb,),
        in_specs=[
            pl.BlockSpec((rb, H), lambda i: (i, 0)),
            pl.BlockSpec((1, rb, 1), lambda i: (i, 0, 0)),
            pl.BlockSpec((B, H), lambda i: (0, 0)),
            pl.BlockSpec((H, H), lambda i: (0, 0)),
            pl.BlockSpec((H, 1), lambda i: (0, 0)),
            pl.BlockSpec((1, 1), lambda i: (0, 0)),
        ],
        out_specs=[pl.BlockSpec((rb, 1), lambda i: (i, 0)),
                   pl.BlockSpec((8, B), lambda i: (0, 0))],
        out_shape=[_f32((n, 1)), _f32((8, B))],
    )(h, batch3, pre, a1h, a2, b2)


def _attn2_body(h_ref, batch_ref, sc_ref, smax_ref, ssum_ref, num_ref, *, rb):
    i = pl.program_id(0)

    @pl.when(i == 0)
    def _():
        ssum_ref[...] = jnp.zeros_like(ssum_ref)
        num_ref[...] = jnp.zeros_like(num_ref)

    bcol = batch_ref[0]
    hb = h_ref[...]
    oh = (bcol == lax.broadcasted_iota(jnp.int32, (rb, B), 1)).astype(F32)
    smg = jnp.dot(oh, smax_ref[:, 0:1], preferred_element_type=F32)
    ex = jnp.exp(sc_ref[...] - smg)
    ssum_ref[:, 0:1] += lax.dot_general(oh, ex, (((0,), (0,)), ((), ())),
                                        preferred_element_type=F32)
    num_ref[...] += lax.dot_general(oh, ex * hb, (((0,), (0,)), ((), ())),
                                    preferred_element_type=F32)


def _attn2_call(h, batch3, scores, smax, rb):
    n = h.shape[0]
    return pl.pallas_call(
        functools.partial(_attn2_body, rb=rb),
        grid=(n // rb,),
        in_specs=[
            pl.BlockSpec((rb, H), lambda i: (i, 0)),
            pl.BlockSpec((1, rb, 1), lambda i: (i, 0, 0)),
            pl.BlockSpec((rb, 1), lambda i: (i, 0)),
            pl.BlockSpec((B, 8), lambda i: (0, 0)),
        ],
        out_specs=[pl.BlockSpec((B, 8), lambda i: (0, 0)),
                   pl.BlockSpec((B, H), lambda i: (0, 0))],
        out_shape=[_f32((B, 8)), _f32((B, H))],
    )(h, batch3, scores, smax)


def _final_body(num_ref, ssum_ref, hterm_ref,
                p1a_ref, p1b_ref, pb1_ref, p2_ref, pb2_ref,
                pmw_ref, pmb_ref, plw_ref, plb_ref,
                mut_ref, lvt_ref, muv_ref, lvv_ref, eps_ref,
                z_ref, mu_ref, lv_ref):
    hvp = num_ref[...] / jnp.maximum(ssum_ref[:, 0:1], 1e-12)
    hp = jnp.dot(hvp, p1a_ref[...], preferred_element_type=F32)
    hp = hp + jnp.dot(hterm_ref[...], p1b_ref[...], preferred_element_type=F32)
    hp = jnp.maximum(hp + pb1_ref[...], 0.0)
    hp = jnp.maximum(jnp.dot(hp, p2_ref[...], preferred_element_type=F32)
                     + pb2_ref[...], 0.0)
    mup = jnp.dot(hp, pmw_ref[...], preferred_element_type=F32) + pmb_ref[...]
    lvp = jnp.dot(hp, plw_ref[...], preferred_element_type=F32) + plb_ref[...]
    mu = jnp.concatenate([mut_ref[...], muv_ref[...], mup], axis=1)
    lv = jnp.concatenate([lvt_ref[...], lvv_ref[...], lvp], axis=1)
    mu_ref[...] = mu
    lv_ref[...] = lv
    z_ref[...] = mu + jnp.exp(0.5 * lv) * eps_ref[...]


def _final_call(numer, ssum, hterm, wdict, mut, lvt, muv, lvv, eps):
    ins = [numer, ssum, hterm,
           wdict['pz_W1a'], wdict['pz_W1b'], wdict['pz_b1'],
           wdict['pz_W2'], wdict['pz_b2'],
           wdict['pz_mu_W'], wdict['pz_mu_b'],
           wdict['pz_lv_W'], wdict['pz_lv_b'],
           mut, lvt, muv, lvv, eps]
    full = lambda a: pl.BlockSpec(a.shape, lambda: tuple(0 for _ in a.shape))
    return pl.pallas_call(
        _final_body,
        in_specs=[full(a) for a in ins],
        out_specs=[pl.BlockSpec((B, 8), lambda: (0, 0))] * 3,
        out_shape=[_f32((B, 8))] * 3,
    )(*ins)


def kernel(x, edge_attr, params, edge_index, batch):
    n = x.shape[0]
    e = edge_attr.shape[0]
    rb = 5000
    assert n % rb == 0, n

    echunk = NSUB * K * SB
    e_pad = ((e + echunk - 1) // echunk) * echunk
    nchunk = NSUB * K
    nrows = ((n + nchunk) // nchunk) * nchunk

    src = edge_index[0].astype(jnp.int32)
    dst = edge_index[1].astype(jnp.int32)
    src_p = jnp.pad(src, (0, e_pad - e))
    dst_p = jnp.pad(dst, (0, e_pad - e), constant_values=n)
    eat_pad = jnp.pad(jnp.transpose(edge_attr), ((0, 0), (0, e_pad - e)))
    ea8 = jnp.transpose(eat_pad.reshape(3, e_pad // 8, 8),
                        (1, 2, 0)).reshape(e_pad // 8, 24)
    batch3 = batch.astype(jnp.int32).reshape(n // rb, rb, 1)

    p = params
    dh = [4, H, H]
    eye8 = jnp.eye(8, dtype=F32)
    w8 = jnp.stack([
        jnp.stack([jnp.kron(eye8, p['msg_W%d' % l][dh[l]:, 16 * q:16 * (q + 1)])
                   for q in range(4)]) for l in range(3)])
    b8 = jnp.stack([
        jnp.stack([jnp.tile(p['msg_b%d' % l][16 * q:16 * (q + 1)], 8)
                   for q in range(4)]) for l in range(3)]).reshape(3, 4, 1, 128)
    whs = [p['msg_W%d' % l][:dh[l]] for l in range(3)]
    uhs = [p['upd_W%d' % l][:dh[l]] for l in range(3)]
    uqs = [[p['upd_W%d' % l][dh[l] + 16 * q:dh[l] + 16 * (q + 1)]
            for q in range(4)] for l in range(3)]
    ubias = [p['upd_b%d' % l].reshape(1, H) for l in range(3)]

    wdict = {
        'topo_W1a': p['topo_W1'][:H], 'topo_W1b': p['topo_W1'][H:],
        'topo_b1': p['topo_b1'].reshape(1, H),
        'topo_W2': p['topo_W2'], 'topo_b2': p['topo_b2'].reshape(1, H // 2),
        'topo_mu_W': p['topo_mu_W'], 'topo_mu_b': p['topo_mu_b'].reshape(1, 2),
        'topo_lv_W': p['topo_lv_W'], 'topo_lv_b': p['topo_lv_b'].reshape(1, 2),
        'val_W': p['val_W'], 'val_b': p['val_b'].reshape(1, H // 2),
        'val_mu_W': p['val_mu_W'], 'val_mu_b': p['val_mu_b'].reshape(1, 2),
        'val_lv_W': p['val_lv_W'], 'val_lv_b': p['val_lv_b'].reshape(1, 2),
        'attn_A1v': p['attn_W1'][H:], 'attn_b1': p['attn_b1'].reshape(1, H),
        'pz_W1a': p['pz_W1'][:H], 'pz_W1b': p['pz_W1'][H:],
        'pz_b1': p['pz_b1'].reshape(1, 2 * H),
        'pz_W2': p['pz_W2'], 'pz_b2': p['pz_b2'].reshape(1, H),
        'pz_mu_W': p['pz_mu_W'], 'pz_mu_b': p['pz_mu_b'].reshape(1, 4),
        'pz_lv_W': p['pz_lv_W'], 'pz_lv_b': p['pz_lv_b'].reshape(1, 4),
    }
    a1h = p['attn_W1'][:H]
    a2 = p['attn_W2']
    ab2 = p['attn_b2'].reshape(1, 1)
    eps = jax.random.normal(jax.random.key(42), (B, 8), dtype=F32)

    ces = _ce_call(ea8, w8, b8)

    hw4 = _split_quarters(_hwf_call(x, whs[0], rb))
    h = x
    for l in range(3):
        agg = _edge_sc_call(hw4, ces[l], src_p, dst_p, n, nrows)
        aggs = [agg[q, :n] for q in range(4)]
        whn = whs[l + 1] if l < 2 else None
        out = _upd_call(h, aggs, uhs[l], uqs[l], ubias[l], whn, rb)
        if l < 2:
            h, hw4 = out[0], _split_quarters(out[1])
        else:
            h = out

    sumh, cnt, maxh, last_row = _pool_call(h, x, batch3, rb)
    last = jnp.transpose(last_row)
    idx = jnp.clip(last_row[:3].reshape(3 * B), 0, n - 1)
    idx = jnp.pad(idx, (0, NCORE * NSUB * 8 - 3 * B)).reshape(NCORE * NSUB, 8)
    gath = _gather_sc_call(h, idx)
    mut, lvt, muv, lvv, hterm, pre = _heads1_call(sumh, cnt, maxh, last,
                                                  gath, wdict)

    scores, smax_row = _attn1_call(h, batch3, pre, a1h, a2, ab2, rb)
    smax = jnp.transpose(smax_row)
    ssum, numer = _attn2_call(h, batch3, scores, smax, rb)
    z, mu, lv = _final_call(numer, ssum, hterm, wdict, mut, lvt, muv, lvv, eps)
    return z, mu, lv


# speedup vs baseline: 1.0725x; 1.0725x over previous
import functools

import jax
import jax.numpy as jnp
from jax import lax
from jax.experimental import pallas as pl
from jax.experimental.pallas import tpu as pltpu
from jax.experimental.pallas import tpu_sc as plsc

B = 64
H = 64

K = 128
SB = 7
NSUB = 16
NCORE = 2

F32 = jnp.float32

_SC_PARAMS = pltpu.CompilerParams(use_tc_tiling_on_sc=False)


def _f32(shape):
    return jax.ShapeDtypeStruct(shape, F32)


def _ce_body(ea8_ref, w8_ref, b8_ref, o0, o1, o2):
    ea = ea8_ref[...]
    outs = (o0, o1, o2)
    for l in range(3):
        for qq in range(4):
            r = jnp.dot(ea, w8_ref[l, qq], preferred_element_type=F32)
            outs[l][qq] = r + b8_ref[l, qq]


def _ce_call(ea8, w8, b8):
    e8 = ea8.shape[0]
    eb8 = 512
    grid = e8 // eb8
    out_spec = pl.BlockSpec((4, eb8, 128), lambda i: (0, i, 0))
    return pl.pallas_call(
        _ce_body,
        grid=(grid,),
        in_specs=[
            pl.BlockSpec((eb8, 24), lambda i: (i, 0)),
            pl.BlockSpec((3, 4, 24, 128), lambda i: (0, 0, 0, 0)),
            pl.BlockSpec((3, 4, 1, 128), lambda i: (0, 0, 0, 0)),
        ],
        out_specs=[out_spec, out_spec, out_spec],
        out_shape=[_f32((4, e8, 128))] * 3,
    )(ea8, w8, b8)


def _hwf_body(x_ref, w_ref, o_ref):
    o_ref[...] = jnp.dot(x_ref[...], w_ref[...], preferred_element_type=F32)


def _hwf_call(x, wh0, rb):
    n = x.shape[0]
    return pl.pallas_call(
        _hwf_body,
        grid=(n // rb,),
        in_specs=[
            pl.BlockSpec((rb, x.shape[1]), lambda i: (i, 0)),
            pl.BlockSpec(wh0.shape, lambda i: (0, 0)),
        ],
        out_specs=pl.BlockSpec((rb, H), lambda i: (i, 0)),
        out_shape=_f32((n, H)),
    )(x, wh0)


def _split_quarters(hwf):
    n = hwf.shape[0]
    return jnp.transpose(hwf.reshape(n, 4, 16), (1, 0, 2)).reshape(4 * n, 16)


def _edge_sc_call(hw4, ce, srcf, dstf, n, nrows):
    total_blocks = srcf.shape[0] // K
    blocks_pw = total_blocks // NSUB
    rows_pw = nrows // NSUB
    nsuper = blocks_pw // SB
    mesh = plsc.VectorSubcoreMesh(core_axis_name="c", subcore_axis_name="s")

    @functools.partial(
        pl.kernel,
        mesh=mesh,
        compiler_params=_SC_PARAMS,
        out_type=_f32((4, nrows, 128)),
        scratch_types=[
            pltpu.VMEM((4, 1, SB * K), jnp.int32),
            pltpu.VMEM((4, 1, SB * K), jnp.int32),
            pltpu.VMEM((2, SB * K, 16), F32),
            pltpu.VMEM((2, SB * (K // 8), 128), F32),
            pltpu.VMEM_SHARED((nrows, 16), F32),
            pltpu.SemaphoreType.DMA,
            pltpu.SemaphoreType.DMA,
            pltpu.SemaphoreType.DMA,
            pltpu.SemaphoreType.DMA,
        ],
    )
    def k(hw_hbm, ce_hbm, srcf_hbm, dstf_hbm, out_hbm, sidx, didx, gbuf, cbuf,
          acc, semi, semg, semc, sems2):
        c = lax.axis_index("c")
        s = lax.axis_index("s")
        zero = jnp.zeros((16,), F32)
        base_blk = s * blocks_pw

        for qq in range(2):
            q_id = 2 * qq + c
            qn = q_id * n

            @pl.loop(0, K)
            def _(r):
                gbuf[0, r, pl.ds(0, 16)] = zero

            @pl.loop(0, rows_pw // K)
            def _(kk):
                pltpu.sync_copy(gbuf.at[0, pl.ds(0, K)],
                                acc.at[pl.ds(s * rows_pw + kk * K, K)])

            plsc.subcore_barrier()

            def issue_idx(ob):
                m = ob % 4
                b0 = base_blk + ob * SB
                pltpu.async_copy(srcf_hbm.at[pl.ds(b0 * K, SB * K)],
                                 sidx.at[m, 0], semi)
                pltpu.async_copy(dstf_hbm.at[pl.ds(b0 * K, SB * K)],
                                 didx.at[m, 0], semi)

            def drain_and_offset_idx(m):
                pltpu.make_async_copy(srcf_hbm.at[pl.ds(0, SB * K)],
                                      sidx.at[0, 0], semi).wait()
                pltpu.make_async_copy(dstf_hbm.at[pl.ds(0, SB * K)],
                                      didx.at[0, 0], semi).wait()
                @pl.loop(0, SB * K // 16)
                def _(t):
                    v = sidx[m, 0, pl.ds(16 * t, 16)]
                    sidx[m, 0, pl.ds(16 * t, 16)] = v + qn

            def fire(ob):
                q = ob % 2
                m = ob % 4
                b0 = base_blk + ob * SB
                pltpu.async_copy(hw_hbm.at[sidx.at[m, 0]], gbuf.at[q], semg)
                pltpu.async_copy(
                    ce_hbm.at[q_id, pl.ds(b0 * (K // 8), SB * (K // 8))],
                    cbuf.at[q], semc)

            def drain_fire(q, m):
                pltpu.make_async_copy(hw_hbm.at[sidx.at[m, 0]],
                                      gbuf.at[q], semg).wait()
                pltpu.make_async_copy(
                    ce_hbm.at[0, pl.ds(0, SB * (K // 8))],
                    cbuf.at[q], semc).wait()

            def drain_scatter(q, m):
                pltpu.make_async_copy(gbuf.at[q], acc.at[didx.at[m, 0]],
                                      sems2).wait()

            issue_idx(0)
            drain_and_offset_idx(0)
            issue_idx(1)
            fire(0)

            @pl.loop(0, nsuper)
            def _(ob):
                q = ob % 2
                m = ob % 4
                drain_fire(q, m)
                for j in range(SB):
                    @pl.loop(0, K // 8)
                    def _(rr):
                        for sub in range(8):
                            v = (gbuf[q, j * K + 8 * rr + sub, pl.ds(0, 16)]
                                 + cbuf[q, j * (K // 8) + rr,
                                        pl.ds(16 * sub, 16)])
                            gbuf[q, j * K + 8 * rr + sub, pl.ds(0, 16)] = (
                                jnp.maximum(v, 0.0))

                pltpu.async_copy(gbuf.at[q], acc.at[didx.at[m, 0]], sems2,
                                 add=True)

                @pl.when(ob > 0)
                def _():
                    drain_scatter((ob + 1) % 2, (ob + 3) % 4)

                @pl.when(ob < nsuper - 1)
                def _():
                    drain_and_offset_idx((ob + 1) % 4)
                    fire(ob + 1)

                @pl.when(ob < nsuper - 2)
                def _():
                    issue_idx(ob + 2)

            drain_scatter((nsuper - 1) % 2, (nsuper - 1) % 4)
            plsc.subcore_barrier()
            pltpu.sync_copy(
                acc.at[pl.ds(s * rows_pw, rows_pw)],
                out_hbm.at[q_id, pl.ds(s * rows_pw, rows_pw), pl.ds(0, 16)])
            plsc.subcore_barrier()

    return k(hw4, ce, srcf, dstf)


def _upd_compute(h_ref, a_refs, uh_ref, u_refs, bias_ref):
    acc = jnp.dot(h_ref[...], uh_ref[...], preferred_element_type=F32)
    for a_ref, u_ref in zip(a_refs, u_refs):
        acc = acc + jnp.dot(a_ref[0][:, :16], u_ref[...],
                            preferred_element_type=F32)
    return jnp.maximum(acc + bias_ref[...], 0.0)


def _upd_body_hw(h_ref, a0, a1, a2, a3, uh_ref, u0, u1, u2, u3, bias_ref,
                 whn_ref, ho_ref, hwo_ref):
    hn = _upd_compute(h_ref, (a0, a1, a2, a3), uh_ref, (u0, u1, u2, u3),
                      bias_ref)
    ho_ref[...] = hn
    hwo_ref[...] = jnp.dot(hn, whn_ref[...], preferred_element_type=F32)


def _upd_body(h_ref, a0, a1, a2, a3, uh_ref, u0, u1, u2, u3, bias_ref,
              ho_ref):
    hn = _upd_compute(h_ref, (a0, a1, a2, a3), uh_ref, (u0, u1, u2, u3),
                      bias_ref)
    ho_ref[:, :H] = hn
    ho_ref[:, H:] = jnp.zeros_like(hn)


def _upd_call(h, agg, uh, us, bias, whn, rb, last):
    n = h.shape[0]
    dh = h.shape[1]
    base_in = ([pl.BlockSpec((rb, dh), lambda i: (i, 0))]
               + [pl.BlockSpec((1, rb, 128), lambda i, q=q: (q, i, 0))
                  for q in range(4)]
               + [pl.BlockSpec(uh.shape, lambda i: (0, 0))]
               + [pl.BlockSpec((16, H), lambda i: (0, 0))] * 4
               + [pl.BlockSpec((1, H), lambda i: (0, 0))])
    args = [h, agg, agg, agg, agg, uh, *us, bias]
    if whn is not None:
        return pl.pallas_call(
            _upd_body_hw,
            grid=(n // rb,),
            in_specs=base_in + [pl.BlockSpec((H, H), lambda i: (0, 0))],
            out_specs=[
                pl.BlockSpec((rb, H), lambda i: (i, 0)),
                pl.BlockSpec((rb, H), lambda i: (i, 0)),
            ],
            out_shape=[_f32((n, H)), _f32((n, H))],
        )(*args, whn)
    return pl.pallas_call(
        _upd_body,
        grid=(n // rb,),
        in_specs=base_in,
        out_specs=pl.BlockSpec((rb, 2 * H), lambda i: (i, 0)),
        out_shape=_f32((n, 2 * H)),
    )(*args)


def _pool_body(h_ref, x_ref, batch_ref, sum_ref, cnt_ref, max_ref, last_ref,
               *, rb):
    i = pl.program_id(0)

    @pl.when(i == 0)
    def _():
        sum_ref[...] = jnp.zeros_like(sum_ref)
        cnt_ref[...] = jnp.zeros_like(cnt_ref)
        max_ref[...] = jnp.full_like(max_ref, -1e30)
        last_ref[...] = jnp.full_like(last_ref, -1)

    bcol = batch_ref[0]
    hb = h_ref[:, :H]
    oh = (bcol == lax.broadcasted_iota(jnp.int32, (rb, B), 1)).astype(F32)
    sum_ref[...] += lax.dot_general(oh, hb, (((0,), (0,)), ((), ())),
                                    preferred_element_type=F32)
    cnt_ref[...] += lax.dot_general(oh, jnp.ones((rb, 8), F32),
                                    (((0,), (0,)), ((), ())),
                                    preferred_element_type=F32)

    def mbody(b, _):
        mask = bcol == b
        m = jnp.max(jnp.where(mask, hb, -1e30), axis=0, keepdims=True)
        cur = max_ref[pl.ds(b, 1), :]
        max_ref[pl.ds(b, 1), :] = jnp.maximum(cur, m)
        return 0

    lax.fori_loop(0, B, mbody, 0)

    xb = x_ref[...]
    best = xb[:, 0:1]
    nt = jnp.zeros_like(best, dtype=jnp.int32)
    for kf in range(1, 4):
        upd = xb[:, kf:kf + 1] > best
        nt = jnp.where(upd, kf, nt)
        best = jnp.maximum(best, xb[:, kf:kf + 1])

    idxg = i * rb + lax.broadcasted_iota(jnp.int32, (rb, B), 0)
    ohb = oh > 0.0
    for t in range(3):
        cond = (nt == t) & ohb
        last_t = jnp.max(jnp.where(cond, idxg, -1), axis=0, keepdims=True)
        cur = last_ref[pl.ds(t, 1), :]
        last_ref[pl.ds(t, 1), :] = jnp.maximum(cur, last_t)


def _pool_call(h, x, batch3, rb):
    n = h.shape[0]
    full = lambda shape: pl.BlockSpec(shape, lambda i: tuple(0 for _ in shape))
    return pl.pallas_call(
        functools.partial(_pool_body, rb=rb),
        grid=(n // rb,),
        in_specs=[
            pl.BlockSpec((rb, 2 * H), lambda i: (i, 0)),
            pl.BlockSpec((rb, 4), lambda i: (i, 0)),
            pl.BlockSpec((1, rb, 1), lambda i: (i, 0, 0)),
        ],
        out_specs=[full((B, H)), full((B, 8)), full((B, H)), full((8, B))],
        out_shape=[_f32((B, H)), _f32((B, 8)), _f32((B, H)),
                   jax.ShapeDtypeStruct((8, B), jnp.int32)],
    )(h, x, batch3)


def _gather_sc_call(h, idx):
    mesh = plsc.VectorSubcoreMesh(core_axis_name="c", subcore_axis_name="s")

    @functools.partial(
        pl.kernel,
        mesh=mesh,
        compiler_params=_SC_PARAMS,
        out_type=_f32((NCORE * NSUB * 8, 2 * H)),
        scratch_types=[
            pltpu.VMEM((8,), jnp.int32),
            pltpu.VMEM((8, 2 * H), F32),
            pltpu.SemaphoreType.DMA,
        ],
    )
    def k(h_hbm, idx_hbm, out_hbm, idxv, rows, sem):
        c = lax.axis_index("c")
        s = lax.axis_index("s")
        w = s * NCORE + c
        pltpu.sync_copy(idx_hbm.at[w], idxv)
        pltpu.async_copy(h_hbm.at[idxv], rows, sem).wait()
        pltpu.sync_copy(rows, out_hbm.at[pl.ds(w * 8, 8)])

    return k(h, idx)


def _heads1_body(sum_ref, cnt_ref, max_ref, last_ref, gath_ref,
                 w1a_ref, w1b_ref, b1_ref, w2_ref, b2_ref,
                 tmw_ref, tmb_ref, tlw_ref, tlb_ref,
                 vw_ref, vb_ref, vmw_ref, vmb_ref, vlw_ref, vlb_ref,
                 a1v_ref, ab1_ref,
                 mut_ref, lvt_ref, muv_ref, lvv_ref, hterm_ref, pre_ref):
    cnt = cnt_ref[:, 0:1]
    meanh = sum_ref[...] / jnp.maximum(cnt, 1.0)
    maxh = jnp.where(cnt > 0.0, max_ref[...], 0.0)
    ht = jnp.dot(meanh, w1a_ref[...], preferred_element_type=F32)
    ht = ht + jnp.dot(maxh, w1b_ref[...], preferred_element_type=F32)
    ht = jnp.maximum(ht + b1_ref[...], 0.0)
    ht = jnp.maximum(jnp.dot(ht, w2_ref[...], preferred_element_type=F32)
                     + b2_ref[...], 0.0)
    mut_ref[...] = jnp.dot(ht, tmw_ref[...], preferred_element_type=F32) + tmb_ref[...]
    lvt_ref[...] = jnp.dot(ht, tlw_ref[...], preferred_element_type=F32) + tlb_ref[...]

    terms = []
    for t in range(3):
        valid = last_ref[:, t:t + 1] >= 0
        terms.append(jnp.where(valid, gath_ref[t * B:(t + 1) * B, :H], 0.0))
    hterm = jnp.concatenate(terms, axis=1)
    hterm_ref[...] = hterm
    hv = jnp.maximum(jnp.dot(hterm, vw_ref[...], preferred_element_type=F32)
                     + vb_ref[...], 0.0)
    muv_ref[...] = jnp.dot(hv, vmw_ref[...], preferred_element_type=F32) + vmb_ref[...]
    lvv_ref[...] = jnp.dot(hv, vlw_ref[...], preferred_element_type=F32) + vlb_ref[...]
    pre_ref[...] = jnp.dot(terms[1], a1v_ref[...], preferred_element_type=F32) + ab1_ref[...]


def _heads1_call(sumh, cnt, maxh, last, gath, wdict):
    ins = [sumh, cnt, maxh, last, gath,
           wdict['topo_W1a'], wdict['topo_W1b'], wdict['topo_b1'],
           wdict['topo_W2'], wdict['topo_b2'],
           wdict['topo_mu_W'], wdict['topo_mu_b'],
           wdict['topo_lv_W'], wdict['topo_lv_b'],
           wdict['val_W'], wdict['val_b'],
           wdict['val_mu_W'], wdict['val_mu_b'],
           wdict['val_lv_W'], wdict['val_lv_b'],
           wdict['attn_A1v'], wdict['attn_b1']]
    full = lambda a: pl.BlockSpec(a.shape, lambda: tuple(0 for _ in a.shape))
    return pl.pallas_call(
        _heads1_body,
        in_specs=[full(a) for a in ins],
        out_specs=[pl.BlockSpec((B, 2), lambda: (0, 0)),
                   pl.BlockSpec((B, 2), lambda: (0, 0)),
                   pl.BlockSpec((B, 2), lambda: (0, 0)),
                   pl.BlockSpec((B, 2), lambda: (0, 0)),
                   pl.BlockSpec((B, 3 * H), lambda: (0, 0)),
                   pl.BlockSpec((B, H), lambda: (0, 0))],
        out_shape=[_f32((B, 2)), _f32((B, 2)), _f32((B, 2)), _f32((B, 2)),
                   _f32((B, 3 * H)), _f32((B, H))],
    )(*ins)


def _attn1_body(h_ref, batch_ref, pre_ref, a1h_ref, a2_ref, b2_ref,
                sc_ref, smax_ref, *, rb):
    i = pl.program_id(0)

    @pl.when(i == 0)
    def _():
        smax_ref[...] = jnp.full_like(smax_ref, -1e30)

    bcol = batch_ref[0]
    hb = h_ref[:, :H]
    oh = (bcol == lax.broadcasted_iota(jnp.int32, (rb, B), 1)).astype(F32)
    t1 = jnp.tanh(jnp.dot(hb, a1h_ref[...], preferred_element_type=F32)
                  + jnp.dot(oh, pre_ref[...], preferred_element_type=F32))
    s = jnp.dot(t1, a2_ref[...], preferred_element_type=F32) + b2_ref[...]
    sc_ref[...] = s
    sm = jnp.max(jnp.where(oh > 0.0, s, -1e30), axis=0, keepdims=True)
    smax_ref[0:1, :] = jnp.maximum(smax_ref[0:1, :], sm)


def _attn1_call(h, batch3, pre, a1h, a2, b2, rb):
    n = h.shape[0]
    return pl.pallas_call(
        functools.partial(_attn1_body, rb=rb),
        grid=(n // rb,),
        in_specs=[
            pl.BlockSpec((rb, 2 * H), lambda i: (i, 0)),
            pl.BlockSpec((1, rb, 1), lambda i: (i, 0, 0)),
            pl.BlockSpec((B, H), lambda i: (0, 0)),
            pl.BlockSpec((H, H), lambda i: (0, 0)),
            pl.BlockSpec((H, 1), lambda i: (0, 0)),
            pl.BlockSpec((1, 1), lambda i: (0, 0)),
        ],
        out_specs=[pl.BlockSpec((rb, 1), lambda i: (i, 0)),
                   pl.BlockSpec((8, B), lambda i: (0, 0))],
        out_shape=[_f32((n, 1)), _f32((8, B))],
    )(h, batch3, pre, a1h, a2, b2)


def _attn2_body(h_ref, batch_ref, sc_ref, smax_ref, ssum_ref, num_ref, *, rb):
    i = pl.program_id(0)

    @pl.when(i == 0)
    def _():
        ssum_ref[...] = jnp.zeros_like(ssum_ref)
        num_ref[...] = jnp.zeros_like(num_ref)

    bcol = batch_ref[0]
    hb = h_ref[:, :H]
    oh = (bcol == lax.broadcasted_iota(jnp.int32, (rb, B), 1)).astype(F32)
    smg = jnp.dot(oh, smax_ref[:, 0:1], preferred_element_type=F32)
    ex = jnp.exp(sc_ref[...] - smg)
    ssum_ref[:, 0:1] += lax.dot_general(oh, ex, (((0,), (0,)), ((), ())),
                                        preferred_element_type=F32)
    num_ref[...] += lax.dot_general(oh, ex * hb, (((0,), (0,)), ((), ())),
                                    preferred_element_type=F32)


def _attn2_call(h, batch3, scores, smax, rb):
    n = h.shape[0]
    return pl.pallas_call(
        functools.partial(_attn2_body, rb=rb),
        grid=(n // rb,),
        in_specs=[
            pl.BlockSpec((rb, 2 * H), lambda i: (i, 0)),
            pl.BlockSpec((1, rb, 1), lambda i: (i, 0, 0)),
            pl.BlockSpec((rb, 1), lambda i: (i, 0)),
            pl.BlockSpec((B, 8), lambda i: (0, 0)),
        ],
        out_specs=[pl.BlockSpec((B, 8), lambda i: (0, 0)),
                   pl.BlockSpec((B, H), lambda i: (0, 0))],
        out_shape=[_f32((B, 8)), _f32((B, H))],
    )(h, batch3, scores, smax)


def _final_body(num_ref, ssum_ref, hterm_ref,
                p1a_ref, p1b_ref, pb1_ref, p2_ref, pb2_ref,
                pmw_ref, pmb_ref, plw_ref, plb_ref,
                mut_ref, lvt_ref, muv_ref, lvv_ref, eps_ref,
                z_ref, mu_ref, lv_ref):
    hvp = num_ref[...] / jnp.maximum(ssum_ref[:, 0:1], 1e-12)
    hp = jnp.dot(hvp, p1a_ref[...], preferred_element_type=F32)
    hp = hp + jnp.dot(hterm_ref[...], p1b_ref[...], preferred_element_type=F32)
    hp = jnp.maximum(hp + pb1_ref[...], 0.0)
    hp = jnp.maximum(jnp.dot(hp, p2_ref[...], preferred_element_type=F32)
                     + pb2_ref[...], 0.0)
    mup = jnp.dot(hp, pmw_ref[...], preferred_element_type=F32) + pmb_ref[...]
    lvp = jnp.dot(hp, plw_ref[...], preferred_element_type=F32) + plb_ref[...]
    mu = jnp.concatenate([mut_ref[...], muv_ref[...], mup], axis=1)
    lv = jnp.concatenate([lvt_ref[...], lvv_ref[...], lvp], axis=1)
    mu_ref[...] = mu
    lv_ref[...] = lv
    z_ref[...] = mu + jnp.exp(0.5 * lv) * eps_ref[...]


def _final_call(numer, ssum, hterm, wdict, mut, lvt, muv, lvv, eps):
    ins = [numer, ssum, hterm,
           wdict['pz_W1a'], wdict['pz_W1b'], wdict['pz_b1'],
           wdict['pz_W2'], wdict['pz_b2'],
           wdict['pz_mu_W'], wdict['pz_mu_b'],
           wdict['pz_lv_W'], wdict['pz_lv_b'],
           mut, lvt, muv, lvv, eps]
    full = lambda a: pl.BlockSpec(a.shape, lambda: tuple(0 for _ in a.shape))
    return pl.pallas_call(
        _final_body,
        in_specs=[full(a) for a in ins],
        out_specs=[pl.BlockSpec((B, 8), lambda: (0, 0))] * 3,
        out_shape=[_f32((B, 8))] * 3,
    )(*ins)


def kernel(x, edge_attr, params, edge_index, batch):
    n = x.shape[0]
    e = edge_attr.shape[0]
    rb = 5000
    assert n % rb == 0, n

    echunk = NSUB * K * SB
    e_pad = ((e + echunk - 1) // echunk) * echunk
    nchunk = NSUB * K
    nrows = ((n + nchunk) // nchunk) * nchunk

    src = edge_index[0].astype(jnp.int32)
    dst = edge_index[1].astype(jnp.int32)
    src_p = jnp.pad(src, (0, e_pad - e))
    dst_p = jnp.pad(dst, (0, e_pad - e), constant_values=n)
    eat_pad = jnp.pad(jnp.transpose(edge_attr), ((0, 0), (0, e_pad - e)))
    ea8 = jnp.transpose(eat_pad.reshape(3, e_pad // 8, 8),
                        (1, 2, 0)).reshape(e_pad // 8, 24)
    batch3 = batch.astype(jnp.int32).reshape(n // rb, rb, 1)

    p = params
    dh = [4, H, H]
    eye8 = jnp.eye(8, dtype=F32)
    w8 = jnp.stack([
        jnp.stack([jnp.kron(eye8, p['msg_W%d' % l][dh[l]:, 16 * q:16 * (q + 1)])
                   for q in range(4)]) for l in range(3)])
    b8 = jnp.stack([
        jnp.stack([jnp.tile(p['msg_b%d' % l][16 * q:16 * (q + 1)], 8)
                   for q in range(4)]) for l in range(3)]).reshape(3, 4, 1, 128)
    whs = [p['msg_W%d' % l][:dh[l]] for l in range(3)]
    uhs = [p['upd_W%d' % l][:dh[l]] for l in range(3)]
    uqs = [[p['upd_W%d' % l][dh[l] + 16 * q:dh[l] + 16 * (q + 1)]
            for q in range(4)] for l in range(3)]
    ubias = [p['upd_b%d' % l].reshape(1, H) for l in range(3)]

    wdict = {
        'topo_W1a': p['topo_W1'][:H], 'topo_W1b': p['topo_W1'][H:],
        'topo_b1': p['topo_b1'].reshape(1, H),
        'topo_W2': p['topo_W2'], 'topo_b2': p['topo_b2'].reshape(1, H // 2),
        'topo_mu_W': p['topo_mu_W'], 'topo_mu_b': p['topo_mu_b'].reshape(1, 2),
        'topo_lv_W': p['topo_lv_W'], 'topo_lv_b': p['topo_lv_b'].reshape(1, 2),
        'val_W': p['val_W'], 'val_b': p['val_b'].reshape(1, H // 2),
        'val_mu_W': p['val_mu_W'], 'val_mu_b': p['val_mu_b'].reshape(1, 2),
        'val_lv_W': p['val_lv_W'], 'val_lv_b': p['val_lv_b'].reshape(1, 2),
        'attn_A1v': p['attn_W1'][H:], 'attn_b1': p['attn_b1'].reshape(1, H),
        'pz_W1a': p['pz_W1'][:H], 'pz_W1b': p['pz_W1'][H:],
        'pz_b1': p['pz_b1'].reshape(1, 2 * H),
        'pz_W2': p['pz_W2'], 'pz_b2': p['pz_b2'].reshape(1, H),
        'pz_mu_W': p['pz_mu_W'], 'pz_mu_b': p['pz_mu_b'].reshape(1, 4),
        'pz_lv_W': p['pz_lv_W'], 'pz_lv_b': p['pz_lv_b'].reshape(1, 4),
    }
    a1h = p['attn_W1'][:H]
    a2 = p['attn_W2']
    ab2 = p['attn_b2'].reshape(1, 1)
    eps = jax.random.normal(jax.random.key(42), (B, 8), dtype=F32)

    ces = _ce_call(ea8, w8, b8)

    hw4 = _split_quarters(_hwf_call(x, whs[0], rb))
    h = x
    for l in range(3):
        agg = _edge_sc_call(hw4, ces[l], src_p, dst_p, n, nrows)
        whn = whs[l + 1] if l < 2 else None
        out = _upd_call(h, agg, uhs[l], uqs[l], ubias[l], whn, rb, l == 2)
        if l < 2:
            h, hw4 = out[0], _split_quarters(out[1])
        else:
            h = out

    sumh, cnt, maxh, last_row = _pool_call(h, x, batch3, rb)
    last = jnp.transpose(last_row)
    idx = jnp.clip(last_row[:3].reshape(3 * B), 0, n - 1)
    idx = jnp.pad(idx, (0, NCORE * NSUB * 8 - 3 * B)).reshape(NCORE * NSUB, 8)
    gath = _gather_sc_call(h, idx)
    mut, lvt, muv, lvv, hterm, pre = _heads1_call(sumh, cnt, maxh, last,
                                                  gath, wdict)

    scores, smax_row = _attn1_call(h, batch3, pre, a1h, a2, ab2, rb)
    smax = jnp.transpose(smax_row)
    ssum, numer = _attn2_call(h, batch3, scores, smax, rb)
    z, mu, lv = _final_call(numer, ssum, hterm, wdict, mut, lvt, muv, lvv, eps)
    return z, mu, lv
